# Initial kernel scaffold; baseline (speedup 1.0000x reference)
import jax
import jax.numpy as jnp
from jax.experimental import pallas as pl


def kernel(x0, edge_attr0, edge_index0, x1, edge_attr1, edge_index1, batch0, batch1, params):
    raise NotImplementedError("write your pallas kernel here")


# speedup vs baseline: 10.0106x; 10.0106x over previous
import functools
import math

import jax
import jax.numpy as jnp
import numpy as np
from jax import lax
from jax.experimental import pallas as pl
from jax.experimental.pallas import tpu as pltpu
from jax.experimental.pallas import tpu_sc as plsc

NC = 2
NS = 16
NW = NC * NS
L = 16
EB = 400
IMIN = np.int32(-2**31)


_SC_PARAMS = pltpu.CompilerParams(needs_layout_passes=False)


def _mesh():
    return plsc.VectorSubcoreMesh(core_axis_name="c", subcore_axis_name="s")


def _pad128(n):
    return ((n + 255) // 256) * 256


@functools.lru_cache(None)
def _k_seg_scalar(n_pad, C, E):
    epw = E // NW
    nb = epw // EB
    spw = n_pad // NS

    def body(tab_hbm, src_hbm, dst_hbm, w_hbm, out_hbm, *refs):
        accs = refs[:C]
        tab_loc, srcb, dstb = refs[C], refs[C + 1], refs[C + 2]
        wbs = refs[C + 3:C + 3 + C]
        cbs = refs[C + 3 + C:C + 3 + 2 * C]
        zb = refs[-1]
        cc = lax.axis_index("c")
        ss = lax.axis_index("s")
        wid = cc * NS + ss
        ebase = wid * epw
        zv = jnp.zeros((L,), jnp.float32)
        for t in range(zb.shape[0] // L):
            zb[pl.ds(t * L, L)] = zv
        for c in range(C):
            pltpu.sync_copy(zb.at[pl.ds(0, spw)], accs[c].at[pl.ds(ss * spw, spw)])
        pltpu.sync_copy(tab_hbm, tab_loc)
        plsc.subcore_barrier()

        def ebody(t, carry):
            e0 = ebase + t * EB
            pltpu.sync_copy(src_hbm.at[pl.ds(e0, EB)], srcb)
            pltpu.sync_copy(dst_hbm.at[pl.ds(e0, EB)], dstb)
            for c in range(C):
                pltpu.sync_copy(w_hbm.at[pl.ds(c * E + e0, EB)], wbs[c])

            def qbody(q, carry2):
                iv = lax.iota(jnp.int32, L) + q * L
                s16 = plsc.load_gather(srcb, [iv])
                tv = plsc.load_gather(tab_loc, [s16])
                for c in range(C):
                    wc = plsc.load_gather(wbs[c], [iv])
                    plsc.store_scatter(cbs[c], [iv], tv * wc)
                return carry2

            lax.fori_loop(0, EB // L, qbody, 0)
            for c in range(C):
                pltpu.sync_copy(cbs[c], accs[c].at[dstb], add=True)
            return carry

        lax.fori_loop(0, nb, ebody, 0)
        plsc.subcore_barrier()
        for c in range(C):
            pltpu.sync_copy(accs[c].at[pl.ds(ss * spw, spw)], zb.at[pl.ds(0, spw)])
            pltpu.sync_copy(zb.at[pl.ds(0, spw)],
                            out_hbm.at[pl.ds((cc * C + c) * n_pad + ss * spw, spw)])

    zlen = ((spw + L - 1) // L) * L
    scratch = ([pltpu.VMEM_SHARED((n_pad,), jnp.float32)] * C
               + [pltpu.VMEM((n_pad,), jnp.float32),
                  pltpu.VMEM((EB,), jnp.int32), pltpu.VMEM((EB,), jnp.int32)]
               + [pltpu.VMEM((EB,), jnp.float32)] * C
               + [pltpu.VMEM((EB,), jnp.float32)] * C
               + [pltpu.VMEM((zlen,), jnp.float32)])
    return pl.kernel(body,
                     out_type=jax.ShapeDtypeStruct((NC * C * n_pad,), jnp.float32),
                     mesh=_mesh(), compiler_params=_SC_PARAMS,
                     scratch_types=scratch)


@functools.lru_cache(None)
def _k_edge(n_tab, n_pad, C, E):
    EB = 80
    epw = E // NW
    nb = epw // EB
    rpt = n_pad // NS
    nz = rpt // 8

    def body(x_hbm, src_hbm, dst_hbm, w_hbm, dis_hbm, out_hbm,
             acc, dis_loc, srcb, dstb, wb, normb, rows, scaled, zrow, sem):
        cc = lax.axis_index("c")
        ss = lax.axis_index("s")
        wid = cc * NS + ss
        ebase = wid * epw
        r0 = ss * rpt
        zv = jnp.zeros((L,), jnp.float32)
        for i in range(8):
            for j in range(8):
                zrow[i, pl.ds(j * L, L)] = zv
        for ch in range(C):
            def zbody(t, carry):
                pltpu.sync_copy(zrow, acc.at[pl.ds(r0 + t * 8, 8)])
                return carry
            lax.fori_loop(0, nz, zbody, 0)
            pltpu.sync_copy(dis_hbm.at[pl.ds(ch * n_pad, n_pad)], dis_loc)
            plsc.subcore_barrier()

            def ebody(t, carry):
                e0 = ebase + t * EB
                pltpu.sync_copy(src_hbm.at[pl.ds(e0, EB)], srcb)
                pltpu.sync_copy(dst_hbm.at[pl.ds(e0, EB)], dstb)
                pltpu.sync_copy(w_hbm.at[pl.ds(ch * E + e0, EB)], wb)

                def qbody(q, carry2):
                    iv = lax.iota(jnp.int32, L) + q * L
                    s16 = plsc.load_gather(srcb, [iv])
                    d16 = plsc.load_gather(dstb, [iv])
                    w16 = plsc.load_gather(wb, [iv])
                    sv = plsc.load_gather(dis_loc, [s16])
                    dv = plsc.load_gather(dis_loc, [d16])
                    plsc.store_scatter(normb, [iv], sv * w16 * dv)
                    return carry2
                lax.fori_loop(0, EB // L, qbody, 0)

                pltpu.async_copy(x_hbm.at[srcb], rows, sem).wait()

                def sbody(r, carry2):
                    rv = jnp.full((L,), r, jnp.int32)
                    ns = plsc.load_gather(normb, [rv])
                    for j in range(8):
                        cv = lax.iota(jnp.int32, L) + j * L
                        v = plsc.load_gather(rows, [rv, cv])
                        plsc.store_scatter(scaled, [rv, cv], v * ns)
                    return carry2
                lax.fori_loop(0, EB, sbody, 0)
                pltpu.sync_copy(scaled, acc.at[dstb], add=True)
                return carry

            lax.fori_loop(0, nb, ebody, 0)
            plsc.subcore_barrier()

            def dbody(t, carry):
                rr = r0 + t * EB
                pltpu.sync_copy(acc.at[pl.ds(rr, EB)], rows)
                pltpu.sync_copy(rows, out_hbm.at[cc, ch, pl.ds(rr, EB)])
                return carry
            lax.fori_loop(0, rpt // EB, dbody, 0)
            plsc.subcore_barrier()

    scratch = [pltpu.VMEM_SHARED((n_pad, 128), jnp.float32),
               pltpu.VMEM((n_pad,), jnp.float32),
               pltpu.VMEM((EB,), jnp.int32), pltpu.VMEM((EB,), jnp.int32),
               pltpu.VMEM((EB,), jnp.float32), pltpu.VMEM((EB,), jnp.float32),
               pltpu.VMEM((EB, 128), jnp.float32), pltpu.VMEM((EB, 128), jnp.float32),
               pltpu.VMEM((8, 128), jnp.float32),
               pltpu.SemaphoreType.DMA]
    return pl.kernel(body,
                     out_type=jax.ShapeDtypeStruct((NC, C, n_pad, 128), jnp.float32),
                     mesh=_mesh(), compiler_params=_SC_PARAMS,
                     scratch_types=scratch)


@functools.lru_cache(None)
def _k_pool(n_pad, k_pad):
    npw = n_pad // NS
    npw16 = ((npw + L - 1) // L) * L
    kpw = k_pad // NS
    rpw = k_pad // NW

    def body(z_hbm, nidx_hbm, px_hbm, perm, nidxb, valsb, permb, rows, zb, sem):
        cc = lax.axis_index("c")
        ss = lax.axis_index("s")
        zvi = jnp.zeros((L,), jnp.int32)
        for t in range(kpw // L):
            zb[pl.ds(t * L, L)] = zvi
        pltpu.sync_copy(zb, perm.at[pl.ds(ss * kpw, kpw)])
        plsc.subcore_barrier()
        nb0 = ss * npw
        pltpu.sync_copy(nidx_hbm.at[pl.ds(nb0, npw)], nidxb)
        for t in range(npw16 // L):
            iv = lax.iota(jnp.int32, L) + t * L
            valsb[pl.ds(t * L, L)] = iv + nb0
        pltpu.sync_copy(valsb.at[pl.ds(0, npw)], perm.at[nidxb])
        plsc.subcore_barrier()
        g0 = cc * (k_pad // NC) + ss * rpw
        pltpu.sync_copy(perm.at[pl.ds(g0, rpw)], permb)
        pltpu.async_copy(z_hbm.at[permb], rows, sem).wait()
        pltpu.sync_copy(rows, px_hbm.at[pl.ds(g0, rpw)])

    scratch = [pltpu.VMEM_SHARED((k_pad,), jnp.int32),
               pltpu.VMEM((npw,), jnp.int32),
               pltpu.VMEM((npw16,), jnp.int32),
               pltpu.VMEM((rpw,), jnp.int32),
               pltpu.VMEM((rpw, 128), jnp.float32),
               pltpu.VMEM((kpw,), jnp.int32),
               pltpu.SemaphoreType.DMA]
    return pl.kernel(body,
                     out_type=jax.ShapeDtypeStruct((k_pad, 128), jnp.float32),
                     mesh=_mesh(), compiler_params=_SC_PARAMS,
                     scratch_types=scratch)


@functools.lru_cache(None)
def _k_remap(n_pad, E, k):
    epw = E // NW
    nb = epw // EB

    def body(nidx_hbm, src_hbm, dst_hbm, w_hbm, so_hbm, do_hbm, vo_hbm,
             nidx_loc, srcb, dstb, wb, sob, dob, vob):
        cc = lax.axis_index("c")
        ss = lax.axis_index("s")
        wid = cc * NS + ss
        ebase = wid * epw
        pltpu.sync_copy(nidx_hbm, nidx_loc)
        kk = jnp.int32(k)

        def ebody(t, carry):
            e0 = ebase + t * EB
            pltpu.sync_copy(src_hbm.at[pl.ds(e0, EB)], srcb)
            pltpu.sync_copy(dst_hbm.at[pl.ds(e0, EB)], dstb)
            pltpu.sync_copy(w_hbm.at[pl.ds(e0, EB)], wb)

            def qbody(q, carry2):
                iv = lax.iota(jnp.int32, L) + q * L
                s16 = plsc.load_gather(srcb, [iv])
                d16 = plsc.load_gather(dstb, [iv])
                w16 = plsc.load_gather(wb, [iv])
                ns = plsc.load_gather(nidx_loc, [s16])
                nd = plsc.load_gather(nidx_loc, [d16])
                valid = (ns < kk) & (nd < kk)
                spread = lax.iota(jnp.int32, L) + (q % 3) * L + kk
                plsc.store_scatter(sob, [iv], jnp.where(valid, ns, spread))
                plsc.store_scatter(dob, [iv], jnp.where(valid, nd, spread))
                plsc.store_scatter(vob, [iv], jnp.where(valid, w16, 0.0))
                return carry2
            lax.fori_loop(0, EB // L, qbody, 0)
            pltpu.sync_copy(sob, so_hbm.at[pl.ds(e0, EB)])
            pltpu.sync_copy(dob, do_hbm.at[pl.ds(e0, EB)])
            pltpu.sync_copy(vob, vo_hbm.at[pl.ds(e0, EB)])
            return carry

        lax.fori_loop(0, nb, ebody, 0)

    scratch = [pltpu.VMEM((n_pad,), jnp.int32),
               pltpu.VMEM((EB,), jnp.int32), pltpu.VMEM((EB,), jnp.int32),
               pltpu.VMEM((EB,), jnp.float32),
               pltpu.VMEM((EB,), jnp.int32), pltpu.VMEM((EB,), jnp.int32),
               pltpu.VMEM((EB,), jnp.float32)]
    return pl.kernel(body,
                     out_type=(jax.ShapeDtypeStruct((E,), jnp.int32),
                               jax.ShapeDtypeStruct((E,), jnp.int32),
                               jax.ShapeDtypeStruct((E,), jnp.float32)),
                     mesh=_mesh(), compiler_params=_SC_PARAMS,
                     scratch_types=scratch)


def _sc_seg_scalar(tab, src, dst, w):
    n_pad = tab.shape[0]
    C, E = w.shape
    out = _k_seg_scalar(n_pad, C, E)(tab, src, dst, w.reshape(-1))
    return out.reshape(NC, C, n_pad)


def _sc_edge(x, src, dst, w, dis):
    C, n_pad = dis.shape
    E = src.shape[0]
    return _k_edge(x.shape[0], n_pad, C, E)(x, src, dst, w.reshape(-1),
                                            dis.reshape(-1))


def _sc_pool(z, nidx, k_pad):
    return _k_pool(z.shape[0], k_pad)(z, nidx)


def _sc_remap(nidx, src, dst, w, k):
    return _k_remap(nidx.shape[0], src.shape[0], k)(nidx, src, dst, w)


@functools.lru_cache(None)
def _t0(C, n_pad):
    def body(degp_ref, dis_ref, inv_ref):
        deg = degp_ref[0] + degp_ref[1] + 1.0
        dis_ref[...] = lax.rsqrt(deg)
        inv_ref[...] = 1.0 / deg
    return pl.pallas_call(
        body,
        out_shape=(jax.ShapeDtypeStruct((C, n_pad), jnp.float32),
                   jax.ShapeDtypeStruct((C, n_pad), jnp.float32)))


@functools.lru_cache(None)
def _t1(C, n_pad, relu):
    def body(part_ref, x_ref, inv_ref, w_ref, b_ref, wrel_ref, g_ref, y_ref):
        i = pl.program_id(0)
        xb = x_ref[...]
        acc = jnp.zeros((128, 128), jnp.float32)
        for c in range(C):
            ivc = inv_ref[c, pl.ds(i * 128, 128)]
            A = part_ref[0, c] + part_ref[1, c] + xb * ivc[:, None]
            h = jnp.dot(A, w_ref[c], preferred_element_type=jnp.float32) + b_ref[c][None, :]
            if relu:
                h = jnp.maximum(h, 0.0)
            acc = acc + h
        g = acc * (1.0 / C)
        g_ref[...] = g
        y_ref[...] = jnp.dot(g, wrel_ref[...], preferred_element_type=jnp.float32)

    grid = (n_pad // 128,)
    return pl.pallas_call(
        body,
        grid=grid,
        in_specs=[
            pl.BlockSpec((NC, C, 128, 128), lambda i: (0, 0, i, 0)),
            pl.BlockSpec((128, 128), lambda i: (i, 0)),
            pl.BlockSpec((C, n_pad), lambda i: (0, 0)),
            pl.BlockSpec((C, 128, 128), lambda i: (0, 0, 0)),
            pl.BlockSpec((C, 128), lambda i: (0, 0)),
            pl.BlockSpec((128, 1), lambda i: (0, 0)),
        ],
        out_specs=[pl.BlockSpec((128, 128), lambda i: (i, 0)),
                   pl.BlockSpec((128, 1), lambda i: (i, 0))],
        out_shape=(jax.ShapeDtypeStruct((n_pad, 128), jnp.float32),
                   jax.ShapeDtypeStruct((n_pad, 1), jnp.float32)))


def _excl_cumsum(m, R):
    M = m.reshape(R, 128)
    rt = jnp.sum(M, axis=1, keepdims=True)
    ia = lax.broadcasted_iota(jnp.int32, (R, R), 0)
    ja = lax.broadcasted_iota(jnp.int32, (R, R), 1)
    tril = (ja < ia).astype(jnp.float32)
    carry = jnp.dot(tril, rt, preferred_element_type=jnp.float32)
    ib = lax.broadcasted_iota(jnp.int32, (128, 128), 0)
    jb = lax.broadcasted_iota(jnp.int32, (128, 128), 1)
    ups = (ib < jb).astype(jnp.float32)
    inner = jnp.dot(M, ups, preferred_element_type=jnp.float32)
    return (carry + inner).reshape(R * 128)


@functools.lru_cache(None)
def _t2(n_pad, n, k):
    R = n_pad // 128

    def body(aggy_ref, brel_ref, wroot_ref, g_ref, z_ref, nidx_ref, gp_ref):
        gg = g_ref[...]
        r = jnp.dot(gg, wroot_ref[...], preferred_element_type=jnp.float32)[:, 0]
        s = jnp.tanh(aggy_ref[0] + aggy_ref[1] + brel_ref[0, 0] + r)
        ii = lax.bitcast_convert_type(s, jnp.int32)
        key = jnp.where(ii < 0, ii ^ np.int32(0x7FFFFFFF), ii)
        gid = lax.broadcasted_iota(jnp.int32, (R, 128), 0) * 128 + \
            lax.broadcasted_iota(jnp.int32, (R, 128), 1)
        key = jnp.where(gid.reshape(n_pad) < n, key, IMIN)

        def tb(b_, t):
            bit = (31 - b_).astype(jnp.int32)
            c = t | lax.shift_left(np.int32(1), bit)
            csig = c ^ IMIN
            cnt = jnp.sum((key >= csig).astype(jnp.int32))
            return jnp.where(cnt >= k, c, t)
        t_u = lax.fori_loop(0, 32, tb, np.int32(0))
        tkey = t_u ^ IMIN
        gt = key > tkey
        eq = key == tkey
        need = k - jnp.sum(gt.astype(jnp.int32))
        eqx = _excl_cumsum(eq.astype(jnp.float32), R)
        keep = gt | (eq & (eqx < need.astype(jnp.float32)))
        kx = _excl_cumsum(keep.astype(jnp.float32), R)
        nidx = jnp.where(keep, kx.astype(jnp.int32), jnp.int32(k))
        z = gg * s[:, None]
        z_ref[...] = z
        nidx_ref[...] = nidx[:, None]
        gp_ref[...] = (jnp.sum(z * keep.astype(jnp.float32)[:, None], axis=0)
                       * (1.0 / k))[None, :]

    return pl.pallas_call(
        body,
        out_shape=(jax.ShapeDtypeStruct((n_pad, 128), jnp.float32),
                   jax.ShapeDtypeStruct((n_pad, 1), jnp.int32),
                   jax.ShapeDtypeStruct((1, 128), jnp.float32)))


@functools.lru_cache(None)
def _t3():
    def body(r_ref, w1, b1, w2, b2, w3, b3, o_ref):
        h = jnp.maximum(jnp.dot(r_ref[...], w1[...],
                                preferred_element_type=jnp.float32) + b1[...], 0.0)
        h = jnp.maximum(jnp.dot(h, w2[...],
                                preferred_element_type=jnp.float32) + b2[...], 0.0)
        o_ref[...] = jnp.dot(h, w3[...],
                             preferred_element_type=jnp.float32) + b3[...]
    return pl.pallas_call(
        body, out_shape=jax.ShapeDtypeStruct((1, 2), jnp.float32))


def _level(xp, src, dst, wT, scorer_w, n, C, Ws, bs, wrel, brel, wroot, relu):
    n_pad = xp.shape[0]
    E = src.shape[0]
    onesN = jnp.ones((n_pad,), jnp.float32)
    degp = _sc_seg_scalar(onesN, src, dst, wT)
    dis, inv = _t0(C, n_pad)(degp)
    part = _sc_edge(xp, src, dst, wT, dis)
    g, y = _t1(C, n_pad, relu)(part, xp, inv, Ws, bs, wrel)
    aggy = _sc_seg_scalar(y.reshape(n_pad), src, dst, scorer_w)
    k = n // 2
    z, nidx, gp = _t2(n_pad, n, k)(aggy.reshape(NC, n_pad),
                                   brel.reshape(1, 1), wroot, g)
    return z, nidx.reshape(n_pad), gp, k


def _branch(x, ea, ei, p):
    src = ei[0]
    dst = ei[1]
    E = src.shape[0]
    n = x.shape[0]
    n_pad = _pad128(n)
    xp = jnp.pad(x, ((0, n_pad - n), (0, 0)))
    onesE = jnp.ones((E,), jnp.float32)

    W1 = jnp.stack([p['W1' + c] for c in 'ABCD'])
    b1 = jnp.stack([p['b1' + c] for c in 'ABCD'])
    z1, nidx1, gp1, k1 = _level(xp, src, dst, ea.T, onesE[None, :], n, 4,
                                W1, b1, p['P1_Wrel'], p['P1_brel'],
                                p['P1_Wroot'], True)
    k1_pad = _pad128(k1)
    px1 = _sc_pool(z1, nidx1, k1_pad)
    s1, d1, v1 = _sc_remap(nidx1, src, dst, onesE, k1)

    W2 = p['W2'][None]
    b2 = p['b2'][None]
    z2, nidx2, gp2, k2 = _level(px1, s1, d1, v1[None, :], v1[None, :], k1, 1,
                                W2, b2, p['P2_Wrel'], p['P2_brel'],
                                p['P2_Wroot'], True)
    k2_pad = _pad128(k2)
    px2 = _sc_pool(z2, nidx2, k2_pad)
    s2, d2, v2 = _sc_remap(nidx2, s1, d1, v1, k2)

    W3 = p['W3'][None]
    b3 = p['b3'][None]
    _, _, gp3, _ = _level(px2, s2, d2, v2[None, :], v2[None, :], k2, 1,
                          W3, b3, p['P3_Wrel'], p['P3_brel'],
                          p['P3_Wroot'], False)
    return jnp.concatenate([gp1, gp2, gp3], axis=1)


def kernel(x0, edge_attr0, edge_index0, x1, edge_attr1, edge_index1,
           batch0, batch1, params):
    p = params
    r0 = _branch(x0, edge_attr0, edge_index0, p)
    r1 = _branch(x1, edge_attr1, edge_index1, p)
    r = jnp.concatenate([r0, r1], axis=1)
    return _t3()(r, p['M1_W'], p['M1_b'][None, :], p['M2_W'],
                 p['M2_b'][None, :], p['M3_W'], p['M3_b'][None, :])
